# baseline (speedup 1.0000x reference)
import jax
import jax.numpy as jnp
from jax import lax
from jax.experimental import pallas as pl
from jax.experimental.pallas import tpu as pltpu
from jax.experimental.pallas import tpu_sc as plsc

_NC = 2
_NS = 16
_NW = _NC * _NS

_NBUF = 4
_CH = 736

_TC_BLK = 2048
_TC_NBLK = 128


def _sc_gather_body(idx_hbm, table_hbm, out_hbm, *refs):
    idx_bufs = refs[0:_NBUF]
    row_bufs = refs[_NBUF:2 * _NBUF]
    si = refs[2 * _NBUF:3 * _NBUF]
    sg = refs[3 * _NBUF:4 * _NBUF]
    so = refs[4 * _NBUF:5 * _NBUF]

    wid = lax.axis_index("s") * _NC + lax.axis_index("c")
    n_sc = out_hbm.shape[0]
    b_per_w = n_sc // _NW
    base = wid * b_per_w
    nch = b_per_w // _CH

    def idx_copy(c, b):
        return pltpu.make_async_copy(
            idx_hbm.at[pl.ds(base + c * _CH, _CH)], idx_bufs[b], si[b])

    def gather_copy(b):
        return pltpu.make_async_copy(
            table_hbm.at[idx_bufs[b]], row_bufs[b], sg[b])

    def out_copy(c, b):
        return pltpu.make_async_copy(
            row_bufs[b], out_hbm.at[pl.ds(base + c * _CH, _CH)], so[b])

    for b in range(_NBUF):
        idx_copy(b, b).start()
    for b in range(_NBUF):
        idx_copy(b, b).wait()
        gather_copy(b).start()

    def outer(o, carry):
        for b in range(_NBUF):
            c = o * _NBUF + b
            gather_copy(b).wait()
            out_copy(c, b).start()

            @pl.when(c + _NBUF < nch)
            def _():
                idx_copy(c + _NBUF, b).start()

            pb = (b - 1) % _NBUF
            pc = c - 1

            @pl.when(jnp.logical_and(pc >= 0, pc + _NBUF < nch))
            def _():
                out_copy(pc, pb).wait()
                idx_copy(pc + _NBUF, pb).wait()
                gather_copy(pb).start()

        return carry

    lax.fori_loop(0, nch // _NBUF, outer, 0)

    for b in range(_NBUF):
        c = nch - _NBUF + b
        out_copy(c, b).wait()


def _tc_gather_body(idx_smem, table_hbm, out_vmem, sem):
    def row(i, carry):
        v = idx_smem[0, 0, i]
        pltpu.make_async_copy(
            table_hbm.at[pl.ds(v, 1)], out_vmem.at[0, pl.ds(i, 1)], sem
        ).start()
        return carry

    lax.fori_loop(0, _TC_BLK, row, 0)
    pltpu.make_async_copy(
        table_hbm.at[pl.ds(0, _TC_BLK)], out_vmem.at[0], sem).wait()


def kernel(x, table):
    b, h = x.shape
    _, d = table.shape
    n = b * h
    flat = x.reshape(n)
    n_tc = _TC_BLK * _TC_NBLK
    n_sc = n - n_tc

    mesh = plsc.VectorSubcoreMesh(core_axis_name="c", subcore_axis_name="s")
    scratch = (
        [pltpu.VMEM((_CH,), jnp.int32) for _ in range(_NBUF)]
        + [pltpu.VMEM((_CH, d), jnp.float32) for _ in range(_NBUF)]
        + [pltpu.SemaphoreType.DMA for _ in range(3 * _NBUF)]
    )
    sc = pl.kernel(
        _sc_gather_body,
        mesh=mesh,
        out_type=jax.ShapeDtypeStruct((n_sc, d), jnp.float32),
        scratch_types=scratch,
        compiler_params=pltpu.CompilerParams(use_tc_tiling_on_sc=False),
    )
    out_sc = sc(flat, table)

    idx3 = flat[n_sc:].reshape(_TC_NBLK, 1, _TC_BLK)
    out_tc = pl.pallas_call(
        _tc_gather_body,
        grid=(_TC_NBLK,),
        in_specs=[
            pl.BlockSpec((1, 1, _TC_BLK), lambda i: (i, 0, 0),
                         memory_space=pltpu.MemorySpace.SMEM),
            pl.BlockSpec(memory_space=pltpu.MemorySpace.HBM),
        ],
        out_specs=pl.BlockSpec((1, _TC_BLK, d), lambda i: (i, 0, 0)),
        out_shape=jax.ShapeDtypeStruct((_TC_NBLK, _TC_BLK, d), jnp.float32),
        scratch_shapes=[pltpu.SemaphoreType.DMA],
    )(idx3, table)

    out = jnp.concatenate([out_sc, out_tc.reshape(n_tc, d)], axis=0)
    return out.reshape(b, h, d)


# speedup vs baseline: 4.9947x; 4.9947x over previous
import jax
import jax.numpy as jnp
from jax import lax
from jax.experimental import pallas as pl
from jax.experimental.pallas import tpu as pltpu
from jax.experimental.pallas import tpu_sc as plsc

_NC = 2
_NS = 16
_NW = _NC * _NS

_NBUF = 4
_CH = 800


def _gather_body(idx_hbm, table_hbm, out_hbm, *refs):
    idx_bufs = refs[0:_NBUF]
    row_bufs = refs[_NBUF:2 * _NBUF]
    si = refs[2 * _NBUF:3 * _NBUF]
    sg = refs[3 * _NBUF:4 * _NBUF]
    so = refs[4 * _NBUF:5 * _NBUF]

    wid = lax.axis_index("s") * _NC + lax.axis_index("c")
    n = idx_hbm.shape[0]
    b_per_w = n // _NW
    base = wid * b_per_w
    nch = b_per_w // _CH

    def idx_copy(c, b):
        return pltpu.make_async_copy(
            idx_hbm.at[pl.ds(base + c * _CH, _CH)], idx_bufs[b], si[b])

    def gather_copy(b):
        return pltpu.make_async_copy(
            table_hbm.at[idx_bufs[b]], row_bufs[b], sg[b])

    def out_copy(c, b):
        return pltpu.make_async_copy(
            row_bufs[b], out_hbm.at[pl.ds(base + c * _CH, _CH)], so[b])

    for b in range(_NBUF):
        idx_copy(b, b).start()
    for b in range(_NBUF):
        idx_copy(b, b).wait()
        gather_copy(b).start()

    def outer(o, carry):
        for b in range(_NBUF):
            c = o * _NBUF + b
            gather_copy(b).wait()
            out_copy(c, b).start()

            @pl.when(c + _NBUF < nch)
            def _():
                idx_copy(c + _NBUF, b).start()

            pb = (b - 1) % _NBUF
            pc = c - 1

            @pl.when(jnp.logical_and(pc >= 0, pc + _NBUF < nch))
            def _():
                out_copy(pc, pb).wait()
                idx_copy(pc + _NBUF, pb).wait()
                gather_copy(pb).start()

        return carry

    lax.fori_loop(0, nch // _NBUF, outer, 0)

    for b in range(_NBUF):
        c = nch - _NBUF + b
        out_copy(c, b).wait()


def kernel(x, table):
    b, h = x.shape
    _, d = table.shape
    n = b * h
    flat = x.reshape(n)

    mesh = plsc.VectorSubcoreMesh(core_axis_name="c", subcore_axis_name="s")
    scratch = (
        [pltpu.VMEM((_CH,), jnp.int32) for _ in range(_NBUF)]
        + [pltpu.VMEM((_CH, d), jnp.float32) for _ in range(_NBUF)]
        + [pltpu.SemaphoreType.DMA for _ in range(3 * _NBUF)]
    )
    f = pl.kernel(
        _gather_body,
        mesh=mesh,
        out_type=jax.ShapeDtypeStruct((n, d), jnp.float32),
        scratch_types=scratch,
        compiler_params=pltpu.CompilerParams(use_tc_tiling_on_sc=False),
    )
    out = f(flat, table)
    return out.reshape(b, h, d)
